# Initial kernel scaffold; baseline (speedup 1.0000x reference)
import jax
import jax.numpy as jnp
from jax.experimental import pallas as pl


def kernel(user_ids, item_ids, user_emb, item_emb, user_bias, item_bias):
    raise NotImplementedError("write your pallas kernel here")


# speedup vs baseline: 1.3745x; 1.3745x over previous
import functools

import jax
import jax.numpy as jnp
from jax import lax
from jax.experimental import pallas as pl
from jax.experimental.pallas import tpu as pltpu
from jax.experimental.pallas import tpu_sc as plsc

B = 16384
D = 128
CHUNK = 128
LANES = 16
NGROUP = D // LANES


def _body(nc, ns, uid, iid, uemb, iemb, ubias, ibias, out,
          uidx_v, iidx_v, urows, irows, ub_v, ib_v, t16, out_v, sem0, sem1):
    nw = nc * ns
    bpw = B // nw
    nchunks = bpw // CHUNK
    wid = lax.axis_index("s") * nc + lax.axis_index("c")
    base = wid * bpw
    sems = (sem0, sem1)
    iota = lax.iota(jnp.int32, LANES)

    pltpu.sync_copy(uid.at[pl.ds(base, bpw)], uidx_v)
    pltpu.sync_copy(iid.at[pl.ds(base, bpw)], iidx_v)

    def issue(c):
        par = c & 1
        uix = uidx_v.at[pl.ds(c * CHUNK, CHUNK)]
        iix = iidx_v.at[pl.ds(c * CHUNK, CHUNK)]
        bsl = pl.ds(c * CHUNK, CHUNK)
        return [
            pltpu.async_copy(uemb.at[uix], urows.at[par], sems[par]),
            pltpu.async_copy(iemb.at[iix], irows.at[par], sems[par]),
            pltpu.async_copy(ubias.at[uix], ub_v.at[bsl], sems[par]),
            pltpu.async_copy(ibias.at[iix], ib_v.at[bsl], sems[par]),
        ]

    def compute(c):
        par = c & 1
        u = urows.at[par]
        v = irows.at[par]

        @pl.loop(0, CHUNK // LANES)
        def _group(g):
            for r in range(LANES):
                b = g * LANES + r
                s = u[b, pl.ds(0, LANES)] * v[b, pl.ds(0, LANES)]
                for j in range(1, NGROUP):
                    s = s + (u[b, pl.ds(j * LANES, LANES)]
                             * v[b, pl.ds(j * LANES, LANES)])
                t16[r, :] = s
            off = pl.ds(c * CHUNK + g * LANES, LANES)
            acc = ub_v[off] + ib_v[off]
            for t in range(LANES):
                acc = acc + plsc.load_gather(
                    t16, [iota, jnp.full((LANES,), t, jnp.int32)])
            out_v[off] = acc

    handles = issue(0)
    for c in range(nchunks):
        nxt = issue(c + 1) if c + 1 < nchunks else None
        for h in handles:
            h.wait()
        compute(c)
        handles = nxt

    pltpu.sync_copy(out_v, out.at[pl.ds(base, bpw)])


@functools.cache
def _build():
    info = plsc.get_sparse_core_info()
    nc, ns = info.num_cores, info.num_subcores
    bpw = B // (nc * ns)
    mesh = plsc.VectorSubcoreMesh(core_axis_name="c", subcore_axis_name="s")
    return pl.kernel(
        functools.partial(_body, nc, ns),
        out_type=jax.ShapeDtypeStruct((B,), jnp.float32),
        mesh=mesh,
        compiler_params=pltpu.CompilerParams(needs_layout_passes=False),
        scratch_types=[
            pltpu.VMEM((bpw,), jnp.int32),
            pltpu.VMEM((bpw,), jnp.int32),
            pltpu.VMEM((2, CHUNK, D), jnp.float32),
            pltpu.VMEM((2, CHUNK, D), jnp.float32),
            pltpu.VMEM((bpw,), jnp.float32),
            pltpu.VMEM((bpw,), jnp.float32),
            pltpu.VMEM((LANES, LANES), jnp.float32),
            pltpu.VMEM((bpw,), jnp.float32),
            pltpu.SemaphoreType.DMA,
            pltpu.SemaphoreType.DMA,
        ],
    )


@jax.jit
def kernel(user_ids, item_ids, user_emb, item_emb, user_bias, item_bias):
    uid = user_ids.astype(jnp.int32)
    iid = item_ids.astype(jnp.int32)
    ub = user_bias.reshape(-1)
    ib = item_bias.reshape(-1)
    return _build()(uid, iid, user_emb, item_emb, ub, ib)
